# baseline (speedup 1.0000x reference)
import functools

import jax
import jax.numpy as jnp
from jax import lax
from jax.experimental import pallas as pl
from jax.experimental.pallas import tpu as pltpu
from jax.experimental.pallas import tpu_sc as plsc

NUM_NODES = 100000
EMB = 64
EMB_U = 16
T = 2
DIM_A = 20
NEIGH = 10
B = 16384

NW = 32
CH = 64
NCH = B // CH
CPW = NCH // NW
ROWS = CH * T * NEIGH
NT = B // 128


def _sc_body(ne_hbm, ntt_hbm, ti_hbm, nn_hbm,
             out_ne, out_t0, out_t1,
             idx_v, rows_v, acc0_v, acc1_v, tidx_v, nerows_v,
             sem_g, sem_ne):
    c = lax.axis_index("c")
    s = lax.axis_index("s")
    wid = s * 2 + c
    g_base = wid * CPW

    gather_descs = [None, None]
    ne_descs = [None, None]

    def stage(g, buf):
        t = g // 2
        h = g % 2
        pltpu.sync_copy(nn_hbm.at[:, t, :, pl.ds(h * CH, CH)], idx_v[buf])

        gather_descs[buf] = [
            pltpu.async_copy(
                ntt_hbm.at[idx_v[buf].at[k, i]],
                rows_v[buf].at[pl.ds((k * T + i) * CH, CH)], sem_g[buf])
            for k in range(NEIGH) for i in range(T)
        ]
        pltpu.sync_copy(ti_hbm.at[pl.ds(g * CH, CH)], tidx_v[buf])
        ne_descs[buf] = pltpu.async_copy(
            ne_hbm.at[tidx_v[buf]], nerows_v[buf], sem_ne[buf])

    def consume(g, buf):
        for d in gather_descs[buf]:
            d.wait()

        def _accum(e, carry):
            r0 = rows_v[buf][e, pl.ds(0, 16)]
            r1 = rows_v[buf][CH + e, pl.ds(16, 16)]
            for k in range(1, NEIGH):
                r0 = r0 + rows_v[buf][(k * T) * CH + e, pl.ds(0, 16)]
                r1 = r1 + rows_v[buf][(k * T + 1) * CH + e, pl.ds(16, 16)]
            acc0_v[e] = r0 * (1.0 / NEIGH)
            acc1_v[e] = r1 * (1.0 / NEIGH)
            return carry

        lax.fori_loop(0, CH, _accum, 0)
        pltpu.sync_copy(acc0_v, out_t0.at[pl.ds(g * CH, CH)])
        pltpu.sync_copy(acc1_v, out_t1.at[pl.ds(g * CH, CH)])
        ne_descs[buf].wait()
        pltpu.sync_copy(nerows_v[buf], out_ne.at[pl.ds(g * CH, CH)])

    stage(g_base, 0)
    for q in range(CPW):
        if q + 1 < CPW:
            stage(g_base + q + 1, (q + 1) % 2)
        consume(g_base + q, q % 2)


@functools.cache
def _build_sc_gather():
    return pl.kernel(
        _sc_body,
        out_type=[
            jax.ShapeDtypeStruct((B, EMB), jnp.float32),
            jax.ShapeDtypeStruct((B, EMB_U), jnp.float32),
            jax.ShapeDtypeStruct((B, EMB_U), jnp.float32),
        ],
        mesh=plsc.VectorSubcoreMesh(core_axis_name="c", subcore_axis_name="s"),
        compiler_params=pltpu.CompilerParams(use_tc_tiling_on_sc=False),
        scratch_types=[
            [pltpu.VMEM((NEIGH, T, CH), jnp.int32) for _ in range(2)],
            [pltpu.VMEM((ROWS, T * EMB_U), jnp.float32) for _ in range(2)],
            pltpu.VMEM((CH, EMB_U), jnp.float32),
            pltpu.VMEM((CH, EMB_U), jnp.float32),
            [pltpu.VMEM((CH,), jnp.int32) for _ in range(2)],
            [pltpu.VMEM((CH, EMB), jnp.float32) for _ in range(2)],
            [pltpu.SemaphoreType.DMA for _ in range(2)],
            [pltpu.SemaphoreType.DMA for _ in range(2)],
        ],
    )


def _tc_body(types_ref, ne_ref, t0_ref, t1_ref, s1_ref, s2_ref,
             w_ref, out_ref):
    nte0 = t0_ref[...]
    nte1 = t1_ref[...]
    is0 = types_ref[...] == 0
    logit = []
    for tt in range(T):
        s1t = s1_ref[tt]
        s2t = s2_ref[tt]
        h0 = jnp.tanh(jnp.dot(nte0, s1t, preferred_element_type=jnp.float32))
        h1 = jnp.tanh(jnp.dot(nte1, s1t, preferred_element_type=jnp.float32))
        logit.append((jnp.sum(h0 * s2t, axis=1, keepdims=True),
                      jnp.sum(h1 * s2t, axis=1, keepdims=True)))
    l0 = jnp.where(is0, logit[0][0], logit[1][0])
    l1 = jnp.where(is0, logit[0][1], logit[1][1])
    m = jnp.maximum(l0, l1)
    e0 = jnp.exp(l0 - m)
    e1 = jnp.exp(l1 - m)
    inv = 1.0 / (e0 + e1)
    comb = (e0 * inv) * nte0 + (e1 * inv) * nte1
    d0 = jnp.dot(comb, w_ref[0], preferred_element_type=jnp.float32)
    d1 = jnp.dot(comb, w_ref[1], preferred_element_type=jnp.float32)
    x = ne_ref[...] + jnp.where(is0, d0, d1)
    sq = jnp.sum(x * x, axis=1, keepdims=True)
    out_ref[...] = x * lax.rsqrt(jnp.maximum(sq, 1e-12))


BS = 2048


def _tc_combine(types2d, ne_g, t0, t1, s1, s2r, w):
    return pl.pallas_call(
        _tc_body,
        grid=(B // BS,),
        in_specs=[
            pl.BlockSpec((BS, 1), lambda i: (i, 0)),
            pl.BlockSpec((BS, EMB), lambda i: (i, 0)),
            pl.BlockSpec((BS, EMB_U), lambda i: (i, 0)),
            pl.BlockSpec((BS, EMB_U), lambda i: (i, 0)),
            pl.BlockSpec((T, EMB_U, DIM_A), lambda i: (0, 0, 0)),
            pl.BlockSpec((T, 1, DIM_A), lambda i: (0, 0, 0)),
            pl.BlockSpec((T, EMB_U, EMB), lambda i: (0, 0, 0)),
        ],
        out_specs=pl.BlockSpec((BS, EMB), lambda i: (i, 0)),
        out_shape=jax.ShapeDtypeStruct((B, EMB), jnp.float32),
    )(types2d, ne_g, t0, t1, s1, s2r, w)


def kernel(node_embeddings, node_type_embeddings, trans_weights,
           trans_weights_s1, trans_weights_s2, train_inputs, train_labels,
           train_types, node_neigh):
    del train_labels
    ntt_tab = node_type_embeddings.reshape(NUM_NODES, T * EMB_U)
    nn_sc = (node_neigh.transpose(2, 0, 1)
             .reshape(NEIGH, NT, 128, T)
             .transpose(0, 1, 3, 2))
    ne_g, t0, t1 = _build_sc_gather()(node_embeddings, ntt_tab,
                                      train_inputs, nn_sc)
    types2d = train_types.reshape(B, 1)
    s2r = jnp.transpose(trans_weights_s2, (0, 2, 1))
    return _tc_combine(types2d, ne_g, t0, t1, trans_weights_s1, s2r,
                       trans_weights)


# speedup vs baseline: 1.2416x; 1.2416x over previous
import functools

import jax
import jax.numpy as jnp
from jax import lax
from jax.experimental import pallas as pl
from jax.experimental.pallas import tpu as pltpu
from jax.experimental.pallas import tpu_sc as plsc

NUM_NODES = 100000
EMB = 64
EMB_U = 16
T = 2
DIM_A = 20
NEIGH = 10
B = 16384

NW = 32
CH = 64
NCH = B // CH
CPW = NCH // NW
ROWS = CH * T * NEIGH
NT = B // 128


def _sc_body(tab_hbm, ti_hbm, nn_hbm, out,
             idx_v, rows_v, comb_v, tidx_v, nerows_v,
             sem_g, sem_ne):
    c = lax.axis_index("c")
    s = lax.axis_index("s")
    wid = s * 2 + c
    g_base = wid * CPW

    gather_descs = [None, None]
    ne_descs = [None, None]

    def stage(g, buf):
        t = g // 2
        h = g % 2
        pltpu.sync_copy(nn_hbm.at[:, t, :, pl.ds(h * CH, CH)], idx_v[buf])

        def _fix(j, carry):
            for k in range(NEIGH):
                for i in range(T):
                    sl = (k, i, pl.ds(j * 16, 16))
                    idx_v[buf][sl] = idx_v[buf][sl] * 4 + 2
            return carry

        lax.fori_loop(0, CH // 16, _fix, 0)

        gather_descs[buf] = [
            pltpu.async_copy(
                tab_hbm.at[idx_v[buf].at[k, i]],
                rows_v[buf].at[pl.ds((k * T + i) * CH, CH)], sem_g[buf])
            for k in range(NEIGH) for i in range(T)
        ]
        pltpu.sync_copy(ti_hbm.at[pl.ds(g * CH, CH)],
                        tidx_v[buf].at[pl.ds(0, CH)])

        def _fix_ti(j, carry):
            lo = (pl.ds(j * 16, 16),)
            hi = (pl.ds(CH + j * 16, 16),)
            tidx_v[buf][hi] = tidx_v[buf][lo] * 4 + 1
            tidx_v[buf][lo] = tidx_v[buf][lo] * 4
            return carry

        lax.fori_loop(0, CH // 16, _fix_ti, 0)
        ne_descs[buf] = [
            pltpu.async_copy(
                tab_hbm.at[tidx_v[buf].at[pl.ds(h2 * CH, CH)]],
                nerows_v[buf].at[pl.ds(h2 * CH, CH)], sem_ne[buf])
            for h2 in range(2)
        ]

    def consume(g, buf):
        for d in gather_descs[buf]:
            d.wait()
        for d in ne_descs[buf]:
            d.wait()

        def _accum(e, carry):
            r0 = rows_v[buf][e, pl.ds(0, 16)]
            r1 = rows_v[buf][CH + e, pl.ds(16, 16)]
            for k in range(1, NEIGH):
                r0 = r0 + rows_v[buf][(k * T) * CH + e, pl.ds(0, 16)]
                r1 = r1 + rows_v[buf][(k * T + 1) * CH + e, pl.ds(16, 16)]
            comb_v[e, pl.ds(0, 16)] = nerows_v[buf][e, pl.ds(0, 16)]
            comb_v[e, pl.ds(16, 16)] = nerows_v[buf][e, pl.ds(16, 16)]
            comb_v[e, pl.ds(32, 16)] = nerows_v[buf][CH + e, pl.ds(0, 16)]
            comb_v[e, pl.ds(48, 16)] = nerows_v[buf][CH + e, pl.ds(16, 16)]
            comb_v[e, pl.ds(64, 16)] = r0 * (1.0 / NEIGH)
            comb_v[e, pl.ds(80, 16)] = r1 * (1.0 / NEIGH)
            return carry

        lax.fori_loop(0, CH, _accum, 0)
        pltpu.sync_copy(comb_v, out.at[pl.ds(g * CH, CH)])

    stage(g_base, 0)
    for q in range(CPW):
        if q + 1 < CPW:
            stage(g_base + q + 1, (q + 1) % 2)
        consume(g_base + q, q % 2)


@functools.cache
def _build_sc_gather():
    return pl.kernel(
        _sc_body,
        out_type=[
            jax.ShapeDtypeStruct((B, 128), jnp.float32),
        ],
        mesh=plsc.VectorSubcoreMesh(core_axis_name="c", subcore_axis_name="s"),
        compiler_params=pltpu.CompilerParams(use_tc_tiling_on_sc=False),
        scratch_types=[
            [pltpu.VMEM((NEIGH, T, CH), jnp.int32) for _ in range(2)],
            [pltpu.VMEM((ROWS, T * EMB_U), jnp.float32) for _ in range(2)],
            pltpu.VMEM((CH, 128), jnp.float32),
            [pltpu.VMEM((2 * CH,), jnp.int32) for _ in range(2)],
            [pltpu.VMEM((2 * CH, 32), jnp.float32) for _ in range(2)],
            [pltpu.SemaphoreType.DMA for _ in range(2)],
            [pltpu.SemaphoreType.DMA for _ in range(2)],
        ],
    )


VB = 2048
NVB = -(-NUM_NODES // VB)


def _tp_body(xu_ref, xe_ref, o_ref):
    o_ref[...] = jnp.concatenate(
        [xe_ref[...].T, xu_ref[...].T,
         jnp.zeros((VB, 32), jnp.float32)], axis=1)


def _tp_tables(ntt_u, ne_u):
    return pl.pallas_call(
        _tp_body,
        grid=(NVB,),
        in_specs=[
            pl.BlockSpec((T * EMB_U, VB), lambda j: (0, j)),
            pl.BlockSpec((EMB, VB), lambda j: (0, j)),
        ],
        out_specs=pl.BlockSpec((VB, 128), lambda j: (j, 0)),
        out_shape=jax.ShapeDtypeStruct((NUM_NODES, 128), jnp.float32),
        compiler_params=pltpu.CompilerParams(
            dimension_semantics=("parallel",)),
    )(ntt_u, ne_u)


def _tc_body(types_ref, data_ref, scat_ref, s2cat_ref, wcat_ref, out_ref):
    data = data_ref[...]
    ne = data[:, 0:EMB]
    nte0 = data[:, 64:64 + EMB_U]
    nte1 = data[:, 80:80 + EMB_U]
    lhs = data[:, 64:64 + 2 * EMB_U]
    is0 = types_ref[...] == 0
    h = jnp.tanh(jnp.dot(lhs, scat_ref[...],
                         preferred_element_type=jnp.float32))
    hs = h * s2cat_ref[...]
    l_b0t0 = jnp.sum(hs[:, 0:DIM_A], axis=1, keepdims=True)
    l_b1t0 = jnp.sum(hs[:, DIM_A:2 * DIM_A], axis=1, keepdims=True)
    l_b0t1 = jnp.sum(hs[:, 2 * DIM_A:3 * DIM_A], axis=1, keepdims=True)
    l_b1t1 = jnp.sum(hs[:, 3 * DIM_A:4 * DIM_A], axis=1, keepdims=True)
    l0 = jnp.where(is0, l_b0t0, l_b0t1)
    l1 = jnp.where(is0, l_b1t0, l_b1t1)
    m = jnp.maximum(l0, l1)
    e0 = jnp.exp(l0 - m)
    e1 = jnp.exp(l1 - m)
    inv = 1.0 / (e0 + e1)
    comb = (e0 * inv) * nte0 + (e1 * inv) * nte1
    d = jnp.dot(comb, wcat_ref[...], preferred_element_type=jnp.float32)
    x = ne + jnp.where(is0, d[:, 0:EMB], d[:, EMB:2 * EMB])
    sq = jnp.sum(x * x, axis=1, keepdims=True)
    out_ref[...] = x * lax.rsqrt(jnp.maximum(sq, 1e-12))


BS = 2048


def _tc_combine(types2d, data, scat, s2cat, wcat):
    return pl.pallas_call(
        _tc_body,
        grid=(B // BS,),
        in_specs=[
            pl.BlockSpec((BS, 1), lambda i: (i, 0)),
            pl.BlockSpec((BS, 128), lambda i: (i, 0)),
            pl.BlockSpec((2 * EMB_U, 4 * DIM_A), lambda i: (0, 0)),
            pl.BlockSpec((1, 4 * DIM_A), lambda i: (0, 0)),
            pl.BlockSpec((EMB_U, 2 * EMB), lambda i: (0, 0)),
        ],
        out_specs=pl.BlockSpec((BS, EMB), lambda i: (i, 0)),
        out_shape=jax.ShapeDtypeStruct((B, EMB), jnp.float32),
        compiler_params=pltpu.CompilerParams(
            dimension_semantics=("parallel",)),
    )(types2d, data, scat, s2cat, wcat)


def kernel(node_embeddings, node_type_embeddings, trans_weights,
           trans_weights_s1, trans_weights_s2, train_inputs, train_labels,
           train_types, node_neigh):
    del train_labels
    ntt_u = node_type_embeddings.transpose(1, 2, 0).reshape(T * EMB_U,
                                                            NUM_NODES)
    ne_u = jnp.transpose(node_embeddings)
    tab = _tp_tables(ntt_u, ne_u).reshape(NUM_NODES * 4, T * EMB_U)
    nn_sc = (node_neigh.transpose(2, 0, 1)
             .reshape(NEIGH, NT, 128, T)
             .transpose(0, 1, 3, 2))
    (data,) = _build_sc_gather()(tab, train_inputs, nn_sc)
    types2d = train_types.reshape(B, 1)
    z = jnp.zeros((EMB_U, DIM_A), jnp.float32)
    scat = jnp.concatenate(
        [jnp.concatenate([trans_weights_s1[0], z, trans_weights_s1[1], z],
                         axis=1),
         jnp.concatenate([z, trans_weights_s1[0], z, trans_weights_s1[1]],
                         axis=1)], axis=0)
    s2cat = jnp.concatenate(
        [trans_weights_s2[0, :, 0], trans_weights_s2[0, :, 0],
         trans_weights_s2[1, :, 0], trans_weights_s2[1, :, 0]]
    ).reshape(1, 4 * DIM_A)
    wcat = jnp.concatenate([trans_weights[0], trans_weights[1]],
                           axis=1)
    return _tc_combine(types2d, data, scat, s2cat, wcat)


# speedup vs baseline: 1.2514x; 1.0079x over previous
import functools

import jax
import jax.numpy as jnp
from jax import lax
from jax.experimental import pallas as pl
from jax.experimental.pallas import tpu as pltpu
from jax.experimental.pallas import tpu_sc as plsc

NUM_NODES = 100000
EMB = 64
EMB_U = 16
T = 2
DIM_A = 20
NEIGH = 10
B = 16384

NW = 32
CH = 64
NCH = B // CH
CPW = NCH // NW
ROWS = CH * T * NEIGH
NT = B // 128


def _sc_body(tab_hbm, ti_hbm, nn_hbm, out,
             idx_v, rows_v, comb_v, tidx_v, nerows_v,
             sem_g, sem_ne):
    c = lax.axis_index("c")
    s = lax.axis_index("s")
    wid = s * 2 + c
    g_base = wid * CPW

    gather_descs = [None, None]
    ne_descs = [None, None]

    def stage(g, buf):
        t = g // 2
        h = g % 2
        pltpu.sync_copy(nn_hbm.at[:, t, :, pl.ds(h * CH, CH)], idx_v[buf])

        def _fix(j, carry):
            for k in range(NEIGH):
                for i in range(T):
                    sl = (k, i, pl.ds(j * 16, 16))
                    idx_v[buf][sl] = idx_v[buf][sl] * 8 + (4 + i)
            return carry

        lax.fori_loop(0, CH // 16, _fix, 0)

        gather_descs[buf] = [
            pltpu.async_copy(
                tab_hbm.at[idx_v[buf].at[k, i]],
                rows_v[buf].at[pl.ds((k * T + i) * CH, CH)], sem_g[buf])
            for k in range(NEIGH) for i in range(T)
        ]
        pltpu.sync_copy(ti_hbm.at[pl.ds(g * CH, CH)],
                        tidx_v[buf].at[pl.ds(0, CH)])

        def _fix_ti(j, carry):
            lo = (pl.ds(j * 16, 16),)
            base = tidx_v[buf][lo] * 8
            for q in range(3, 0, -1):
                tidx_v[buf][(pl.ds(q * CH + j * 16, 16),)] = base + q
            tidx_v[buf][lo] = base
            return carry

        lax.fori_loop(0, CH // 16, _fix_ti, 0)
        ne_descs[buf] = [
            pltpu.async_copy(
                tab_hbm.at[tidx_v[buf].at[pl.ds(h2 * CH, CH)]],
                nerows_v[buf].at[pl.ds(h2 * CH, CH)], sem_ne[buf])
            for h2 in range(4)
        ]

    def consume(g, buf):
        for d in gather_descs[buf]:
            d.wait()
        for d in ne_descs[buf]:
            d.wait()

        def _accum(e, carry):
            r0 = rows_v[buf][e, :]
            r1 = rows_v[buf][CH + e, :]
            for k in range(1, NEIGH):
                r0 = r0 + rows_v[buf][(k * T) * CH + e, :]
                r1 = r1 + rows_v[buf][(k * T + 1) * CH + e, :]
            for q in range(4):
                comb_v[e, pl.ds(q * 16, 16)] = nerows_v[buf][q * CH + e, :]
            comb_v[e, pl.ds(64, 16)] = r0 * (1.0 / NEIGH)
            comb_v[e, pl.ds(80, 16)] = r1 * (1.0 / NEIGH)
            return carry

        lax.fori_loop(0, CH, _accum, 0)
        pltpu.sync_copy(comb_v, out.at[pl.ds(g * CH, CH)])

    stage(g_base, 0)
    for q in range(CPW):
        if q + 1 < CPW:
            stage(g_base + q + 1, (q + 1) % 2)
        consume(g_base + q, q % 2)


@functools.cache
def _build_sc_gather():
    return pl.kernel(
        _sc_body,
        out_type=[
            jax.ShapeDtypeStruct((B, 128), jnp.float32),
        ],
        mesh=plsc.VectorSubcoreMesh(core_axis_name="c", subcore_axis_name="s"),
        compiler_params=pltpu.CompilerParams(use_tc_tiling_on_sc=False),
        scratch_types=[
            [pltpu.VMEM((NEIGH, T, CH), jnp.int32) for _ in range(2)],
            [pltpu.VMEM((ROWS, EMB_U), jnp.float32) for _ in range(2)],
            pltpu.VMEM((CH, 128), jnp.float32),
            [pltpu.VMEM((4 * CH,), jnp.int32) for _ in range(2)],
            [pltpu.VMEM((4 * CH, EMB_U), jnp.float32) for _ in range(2)],
            [pltpu.SemaphoreType.DMA for _ in range(2)],
            [pltpu.SemaphoreType.DMA for _ in range(2)],
        ],
    )


VB = 2048
NVB = -(-NUM_NODES // VB)


def _tp_body(xu_ref, xe_ref, o_ref):
    o_ref[...] = jnp.concatenate(
        [xe_ref[...].T, xu_ref[...].T,
         jnp.zeros((VB, 32), jnp.float32)], axis=1)


def _tp_tables(ntt_u, ne_u):
    return pl.pallas_call(
        _tp_body,
        grid=(NVB,),
        in_specs=[
            pl.BlockSpec((T * EMB_U, VB), lambda j: (0, j)),
            pl.BlockSpec((EMB, VB), lambda j: (0, j)),
        ],
        out_specs=pl.BlockSpec((VB, 128), lambda j: (j, 0)),
        out_shape=jax.ShapeDtypeStruct((NUM_NODES, 128), jnp.float32),
        compiler_params=pltpu.CompilerParams(
            dimension_semantics=("parallel",)),
    )(ntt_u, ne_u)


def _tc_body(types_ref, data_ref, scat_ref, s2cat_ref, wcat_ref, out_ref):
    data = data_ref[...]
    ne = data[:, 0:EMB]
    nte0 = data[:, 64:64 + EMB_U]
    nte1 = data[:, 80:80 + EMB_U]
    lhs = data[:, 64:64 + 2 * EMB_U]
    is0 = types_ref[...] == 0
    h = jnp.tanh(jnp.dot(lhs, scat_ref[...],
                         preferred_element_type=jnp.float32))
    hs = h * s2cat_ref[...]
    l_b0t0 = jnp.sum(hs[:, 0:DIM_A], axis=1, keepdims=True)
    l_b1t0 = jnp.sum(hs[:, DIM_A:2 * DIM_A], axis=1, keepdims=True)
    l_b0t1 = jnp.sum(hs[:, 2 * DIM_A:3 * DIM_A], axis=1, keepdims=True)
    l_b1t1 = jnp.sum(hs[:, 3 * DIM_A:4 * DIM_A], axis=1, keepdims=True)
    l0 = jnp.where(is0, l_b0t0, l_b0t1)
    l1 = jnp.where(is0, l_b1t0, l_b1t1)
    m = jnp.maximum(l0, l1)
    e0 = jnp.exp(l0 - m)
    e1 = jnp.exp(l1 - m)
    inv = 1.0 / (e0 + e1)
    comb = (e0 * inv) * nte0 + (e1 * inv) * nte1
    d = jnp.dot(comb, wcat_ref[...], preferred_element_type=jnp.float32)
    x = ne + jnp.where(is0, d[:, 0:EMB], d[:, EMB:2 * EMB])
    sq = jnp.sum(x * x, axis=1, keepdims=True)
    out_ref[...] = x * lax.rsqrt(jnp.maximum(sq, 1e-12))


BS = 2048


def _tc_combine(types2d, data, scat, s2cat, wcat):
    return pl.pallas_call(
        _tc_body,
        grid=(B // BS,),
        in_specs=[
            pl.BlockSpec((BS, 1), lambda i: (i, 0)),
            pl.BlockSpec((BS, 128), lambda i: (i, 0)),
            pl.BlockSpec((2 * EMB_U, 4 * DIM_A), lambda i: (0, 0)),
            pl.BlockSpec((1, 4 * DIM_A), lambda i: (0, 0)),
            pl.BlockSpec((EMB_U, 2 * EMB), lambda i: (0, 0)),
        ],
        out_specs=pl.BlockSpec((BS, EMB), lambda i: (i, 0)),
        out_shape=jax.ShapeDtypeStruct((B, EMB), jnp.float32),
        compiler_params=pltpu.CompilerParams(
            dimension_semantics=("parallel",)),
    )(types2d, data, scat, s2cat, wcat)


def kernel(node_embeddings, node_type_embeddings, trans_weights,
           trans_weights_s1, trans_weights_s2, train_inputs, train_labels,
           train_types, node_neigh):
    del train_labels
    ntt_u = node_type_embeddings.transpose(1, 2, 0).reshape(T * EMB_U,
                                                            NUM_NODES)
    ne_u = jnp.transpose(node_embeddings)
    tab = _tp_tables(ntt_u, ne_u).reshape(NUM_NODES * 8, EMB_U)
    nn_sc = (node_neigh.transpose(2, 0, 1)
             .reshape(NEIGH, NT, 128, T)
             .transpose(0, 1, 3, 2))
    (data,) = _build_sc_gather()(tab, train_inputs, nn_sc)
    types2d = train_types.reshape(B, 1)
    z = jnp.zeros((EMB_U, DIM_A), jnp.float32)
    scat = jnp.concatenate(
        [jnp.concatenate([trans_weights_s1[0], z, trans_weights_s1[1], z],
                         axis=1),
         jnp.concatenate([z, trans_weights_s1[0], z, trans_weights_s1[1]],
                         axis=1)], axis=0)
    s2cat = jnp.concatenate(
        [trans_weights_s2[0, :, 0], trans_weights_s2[0, :, 0],
         trans_weights_s2[1, :, 0], trans_weights_s2[1, :, 0]]
    ).reshape(1, 4 * DIM_A)
    wcat = jnp.concatenate([trans_weights[0], trans_weights[1]],
                           axis=1)
    return _tc_combine(types2d, data, scat, s2cat, wcat)


# speedup vs baseline: 1.3346x; 1.0665x over previous
import functools

import jax
import jax.numpy as jnp
from jax import lax
from jax.experimental import pallas as pl
from jax.experimental.pallas import tpu as pltpu
from jax.experimental.pallas import tpu_sc as plsc

NUM_NODES = 100000
EMB = 64
EMB_U = 16
T = 2
DIM_A = 20
NEIGH = 10
B = 16384

NW = 32
CH = 64
NCH = B // CH
CPW = NCH // NW
ROWS = CH * T * NEIGH
NT = B // 128


def _sc_body(tab_hbm, ti_hbm, nn_hbm, out,
             idx_v, rows_v, comb_v, tidx_v, nerows_v,
             sem_g, sem_ne):
    c = lax.axis_index("c")
    s = lax.axis_index("s")
    wid = s * 2 + c
    g_base = wid * CPW

    gather_descs = [None, None]
    ne_descs = [None, None]

    def stage(g, buf):
        t = g // 2
        h = g % 2
        pltpu.sync_copy(nn_hbm.at[:, t, :, pl.ds(h * CH, CH)], idx_v[buf])

        def _fix(j, carry):
            for k in range(NEIGH):
                for i in range(T):
                    sl = (k, i, pl.ds(j * 16, 16))
                    idx_v[buf][sl] = idx_v[buf][sl] * 8 + (4 + i)
            return carry

        lax.fori_loop(0, CH // 16, _fix, 0)

        gather_descs[buf] = [
            pltpu.async_copy(
                tab_hbm.at[idx_v[buf].at[k, i]],
                rows_v[buf].at[pl.ds((k * T + i) * CH, CH)], sem_g[buf])
            for k in range(NEIGH) for i in range(T)
        ]
        pltpu.sync_copy(ti_hbm.at[pl.ds(g * CH, CH)],
                        tidx_v[buf].at[pl.ds(0, CH)])

        def _fix_ti(j, carry):
            lo = (pl.ds(j * 16, 16),)
            base = tidx_v[buf][lo] * 8
            for q in range(3, 0, -1):
                tidx_v[buf][(pl.ds(q * CH + j * 16, 16),)] = base + q
            tidx_v[buf][lo] = base
            return carry

        lax.fori_loop(0, CH // 16, _fix_ti, 0)
        ne_descs[buf] = [
            pltpu.async_copy(
                tab_hbm.at[tidx_v[buf].at[pl.ds(h2 * CH, CH)]],
                nerows_v[buf].at[pl.ds(h2 * CH, CH)], sem_ne[buf])
            for h2 in range(4)
        ]

    def consume(g, buf):
        for d in gather_descs[buf]:
            d.wait()
        for d in ne_descs[buf]:
            d.wait()

        def _accum(e, carry):
            r0 = rows_v[buf][e, :]
            r1 = rows_v[buf][CH + e, :]
            for k in range(1, NEIGH):
                r0 = r0 + rows_v[buf][(k * T) * CH + e, :]
                r1 = r1 + rows_v[buf][(k * T + 1) * CH + e, :]
            for q in range(4):
                comb_v[e, pl.ds(q * 16, 16)] = nerows_v[buf][q * CH + e, :]
            comb_v[e, pl.ds(64, 16)] = r0 * (1.0 / NEIGH)
            comb_v[e, pl.ds(80, 16)] = r1 * (1.0 / NEIGH)
            return carry

        lax.fori_loop(0, CH, _accum, 0)
        pltpu.sync_copy(comb_v, out.at[pl.ds(g * CH, CH)])

    stage(g_base, 0)
    for q in range(CPW):
        if q + 1 < CPW:
            stage(g_base + q + 1, (q + 1) % 2)
        consume(g_base + q, q % 2)


@functools.cache
def _build_sc_gather():
    return pl.kernel(
        _sc_body,
        out_type=[
            jax.ShapeDtypeStruct((B, 128), jnp.float32),
        ],
        mesh=plsc.VectorSubcoreMesh(core_axis_name="c", subcore_axis_name="s"),
        compiler_params=pltpu.CompilerParams(use_tc_tiling_on_sc=False),
        scratch_types=[
            [pltpu.VMEM((NEIGH, T, CH), jnp.int32) for _ in range(2)],
            [pltpu.VMEM((ROWS, EMB_U), jnp.float32) for _ in range(2)],
            pltpu.VMEM((CH, 128), jnp.float32),
            [pltpu.VMEM((4 * CH,), jnp.int32) for _ in range(2)],
            [pltpu.VMEM((4 * CH, EMB_U), jnp.float32) for _ in range(2)],
            [pltpu.SemaphoreType.DMA for _ in range(2)],
            [pltpu.SemaphoreType.DMA for _ in range(2)],
        ],
    )


VB = 2048
NVB = -(-NUM_NODES // VB)


def _tp_body(xu_ref, xe_ref, o_ref):
    o_ref[...] = jnp.concatenate(
        [xe_ref[...].T, xu_ref[...].T,
         jnp.zeros((VB, 32), jnp.float32)], axis=1)


def _tp_tables(ntt_u, ne_u):
    return pl.pallas_call(
        _tp_body,
        grid=(NVB,),
        in_specs=[
            pl.BlockSpec((T * EMB_U, VB), lambda j: (0, j)),
            pl.BlockSpec((EMB, VB), lambda j: (0, j)),
        ],
        out_specs=pl.BlockSpec((VB, 128), lambda j: (j, 0)),
        out_shape=jax.ShapeDtypeStruct((NUM_NODES, 128), jnp.float32),
        compiler_params=pltpu.CompilerParams(
            dimension_semantics=("parallel",)),
    )(ntt_u, ne_u)


def _tc_body(types_ref, data_ref, scat_ref, s2cat_ref, wcat_ref, out_ref):
    data = data_ref[...]
    ne = data[:, 0:EMB]
    nte0 = data[:, 64:64 + EMB_U]
    nte1 = data[:, 80:80 + EMB_U]
    lhs = data[:, 64:64 + 2 * EMB_U]
    is0 = types_ref[...] == 0
    z = jnp.dot(lhs, scat_ref[...], preferred_element_type=jnp.float32)
    zsel = jnp.where(is0, z[:, 0:2 * DIM_A], z[:, 2 * DIM_A:4 * DIM_A])
    s2sel = jnp.where(is0, s2cat_ref[:, 0:2 * DIM_A],
                      s2cat_ref[:, 2 * DIM_A:4 * DIM_A])
    hs = jnp.tanh(zsel) * s2sel
    l0 = jnp.sum(hs[:, 0:DIM_A], axis=1, keepdims=True)
    l1 = jnp.sum(hs[:, DIM_A:2 * DIM_A], axis=1, keepdims=True)
    a0 = 0.5 * (jnp.tanh(0.5 * (l0 - l1)) + 1.0)
    comb = nte1 + a0 * (nte0 - nte1)
    d = jnp.dot(comb, wcat_ref[...], preferred_element_type=jnp.float32)
    x = ne + jnp.where(is0, d[:, 0:EMB], d[:, EMB:2 * EMB])
    sq = jnp.sum(x * x, axis=1, keepdims=True)
    out_ref[...] = x * lax.rsqrt(jnp.maximum(sq, 1e-12))


BS = 2048


def _tc_combine(types2d, data, scat, s2cat, wcat):
    return pl.pallas_call(
        _tc_body,
        grid=(B // BS,),
        in_specs=[
            pl.BlockSpec((BS, 1), lambda i: (i, 0)),
            pl.BlockSpec((BS, 128), lambda i: (i, 0)),
            pl.BlockSpec((2 * EMB_U, 4 * DIM_A), lambda i: (0, 0)),
            pl.BlockSpec((1, 4 * DIM_A), lambda i: (0, 0)),
            pl.BlockSpec((EMB_U, 2 * EMB), lambda i: (0, 0)),
        ],
        out_specs=pl.BlockSpec((BS, EMB), lambda i: (i, 0)),
        out_shape=jax.ShapeDtypeStruct((B, EMB), jnp.float32),
        compiler_params=pltpu.CompilerParams(
            dimension_semantics=("parallel",)),
    )(types2d, data, scat, s2cat, wcat)


def kernel(node_embeddings, node_type_embeddings, trans_weights,
           trans_weights_s1, trans_weights_s2, train_inputs, train_labels,
           train_types, node_neigh):
    del train_labels
    ntt_u = node_type_embeddings.transpose(1, 2, 0).reshape(T * EMB_U,
                                                            NUM_NODES)
    ne_u = jnp.transpose(node_embeddings)
    tab = _tp_tables(ntt_u, ne_u).reshape(NUM_NODES * 8, EMB_U)
    nn_sc = (node_neigh.transpose(2, 0, 1)
             .reshape(NEIGH, NT, 128, T)
             .transpose(0, 1, 3, 2))
    (data,) = _build_sc_gather()(tab, train_inputs, nn_sc)
    types2d = train_types.reshape(B, 1)
    z = jnp.zeros((EMB_U, DIM_A), jnp.float32)
    scat = jnp.concatenate(
        [jnp.concatenate([trans_weights_s1[0], z, trans_weights_s1[1], z],
                         axis=1),
         jnp.concatenate([z, trans_weights_s1[0], z, trans_weights_s1[1]],
                         axis=1)], axis=0)
    s2cat = jnp.concatenate(
        [trans_weights_s2[0, :, 0], trans_weights_s2[0, :, 0],
         trans_weights_s2[1, :, 0], trans_weights_s2[1, :, 0]]
    ).reshape(1, 4 * DIM_A)
    wcat = jnp.concatenate([trans_weights[0], trans_weights[1]],
                           axis=1)
    return _tc_combine(types2d, data, scat, s2cat, wcat)


# speedup vs baseline: 1.4543x; 1.0897x over previous
import functools

import jax
import jax.numpy as jnp
from jax import lax
from jax.experimental import pallas as pl
from jax.experimental.pallas import tpu as pltpu
from jax.experimental.pallas import tpu_sc as plsc

NUM_NODES = 100000
EMB = 64
EMB_U = 16
T = 2
DIM_A = 20
NEIGH = 10
B = 16384

NW = 32
CH = 64
NCH = B // CH
CPW = NCH // NW
ROWS = CH * T * NEIGH
NT = B // 128


def _sc_body(tab_hbm, ti_hbm, nn_hbm, out,
             idx_v, rows_v, comb_v, tidx_v, nerows_v,
             sem_g, sem_ne):
    c = lax.axis_index("c")
    s = lax.axis_index("s")
    wid = s * 2 + c
    g_base = wid * CPW

    gather_descs = [None, None]
    ne_descs = [None, None]

    def stage(g, buf):
        t = g // 2
        h = g % 2
        pltpu.sync_copy(nn_hbm.at[:, t, :, pl.ds(h * CH, CH)], idx_v[buf])

        def _fix(j, carry):
            for k in range(NEIGH):
                for i in range(T):
                    sl = (k, i, pl.ds(j * 16, 16))
                    idx_v[buf][sl] = idx_v[buf][sl] * 8 + (4 + i)
            return carry

        lax.fori_loop(0, CH // 16, _fix, 0)

        gather_descs[buf] = [
            pltpu.async_copy(
                tab_hbm.at[idx_v[buf].at[k, i]],
                rows_v[buf].at[pl.ds((k * T + i) * CH, CH)], sem_g[buf])
            for k in range(NEIGH) for i in range(T)
        ]
        pltpu.sync_copy(ti_hbm.at[pl.ds(g * CH, CH)],
                        tidx_v[buf].at[pl.ds(0, CH)])

        def _fix_ti(j, carry):
            lo = (pl.ds(j * 16, 16),)
            base = tidx_v[buf][lo] * 8
            for q in range(3, 0, -1):
                tidx_v[buf][(pl.ds(q * CH + j * 16, 16),)] = base + q
            tidx_v[buf][lo] = base
            return carry

        lax.fori_loop(0, CH // 16, _fix_ti, 0)
        ne_descs[buf] = [
            pltpu.async_copy(
                tab_hbm.at[tidx_v[buf].at[pl.ds(h2 * CH, CH)]],
                nerows_v[buf].at[pl.ds(h2 * CH, CH)], sem_ne[buf])
            for h2 in range(4)
        ]

    def consume(g, buf):
        for d in gather_descs[buf]:
            d.wait()
        for d in ne_descs[buf]:
            d.wait()

        def _accum(e, carry):
            r0 = rows_v[buf][e, :]
            r1 = rows_v[buf][CH + e, :]
            for k in range(1, NEIGH):
                r0 = r0 + rows_v[buf][(k * T) * CH + e, :]
                r1 = r1 + rows_v[buf][(k * T + 1) * CH + e, :]
            for q in range(4):
                comb_v[e, pl.ds(q * 16, 16)] = nerows_v[buf][q * CH + e, :]
            comb_v[e, pl.ds(64, 16)] = r0 * (1.0 / NEIGH)
            comb_v[e, pl.ds(80, 16)] = r1 * (1.0 / NEIGH)
            return carry

        lax.fori_loop(0, CH, _accum, 0)
        pltpu.sync_copy(comb_v, out.at[pl.ds(g * CH, CH)])

    stage(g_base, 0)
    for q in range(CPW):
        if q + 1 < CPW:
            stage(g_base + q + 1, (q + 1) % 2)
        consume(g_base + q, q % 2)


@functools.cache
def _build_sc_gather():
    return pl.kernel(
        _sc_body,
        out_type=[
            jax.ShapeDtypeStruct((B, 128), jnp.float32),
        ],
        mesh=plsc.VectorSubcoreMesh(core_axis_name="c", subcore_axis_name="s"),
        compiler_params=pltpu.CompilerParams(use_tc_tiling_on_sc=False),
        scratch_types=[
            [pltpu.VMEM((NEIGH, T, CH), jnp.int32) for _ in range(2)],
            [pltpu.VMEM((ROWS, EMB_U), jnp.float32) for _ in range(2)],
            pltpu.VMEM((CH, 128), jnp.float32),
            [pltpu.VMEM((4 * CH,), jnp.int32) for _ in range(2)],
            [pltpu.VMEM((4 * CH, EMB_U), jnp.float32) for _ in range(2)],
            [pltpu.SemaphoreType.DMA for _ in range(2)],
            [pltpu.SemaphoreType.DMA for _ in range(2)],
        ],
    )


VB = 4096
NVB = -(-NUM_NODES // VB)


def _tp_body(xu_ref, xe_ref, o_ref):
    o_ref[...] = jnp.concatenate(
        [xe_ref[...].T, xu_ref[...].T,
         jnp.zeros((VB, 32), jnp.float32)], axis=1)


def _tp_tables(ntt_u, ne_u):
    return pl.pallas_call(
        _tp_body,
        grid=(NVB,),
        in_specs=[
            pl.BlockSpec((T * EMB_U, VB), lambda j: (0, j)),
            pl.BlockSpec((EMB, VB), lambda j: (0, j)),
        ],
        out_specs=pl.BlockSpec((VB, 128), lambda j: (j, 0)),
        out_shape=jax.ShapeDtypeStruct((NUM_NODES, 128), jnp.float32),
        compiler_params=pltpu.CompilerParams(
            dimension_semantics=("parallel",)),
    )(ntt_u, ne_u)


def _tc_body(types_ref, data_ref, scat_ref, s2cat_ref, wcat_ref, out_ref):
    data = data_ref[...]
    ne = data[:, 0:EMB]
    nte0 = data[:, 64:64 + EMB_U]
    nte1 = data[:, 80:80 + EMB_U]
    lhs = data[:, 64:64 + 2 * EMB_U]
    is0 = types_ref[...] == 0
    z = jnp.dot(lhs, scat_ref[...], preferred_element_type=jnp.float32)
    zsel = jnp.where(is0, z[:, 0:2 * DIM_A], z[:, 2 * DIM_A:4 * DIM_A])
    s2sel = jnp.where(is0, s2cat_ref[:, 0:2 * DIM_A],
                      s2cat_ref[:, 2 * DIM_A:4 * DIM_A])
    hs = jnp.tanh(zsel) * s2sel
    l0 = jnp.sum(hs[:, 0:DIM_A], axis=1, keepdims=True)
    l1 = jnp.sum(hs[:, DIM_A:2 * DIM_A], axis=1, keepdims=True)
    a0 = 0.5 * (jnp.tanh(0.5 * (l0 - l1)) + 1.0)
    comb = nte1 + a0 * (nte0 - nte1)
    d = jnp.dot(comb, wcat_ref[...], preferred_element_type=jnp.float32)
    x = ne + jnp.where(is0, d[:, 0:EMB], d[:, EMB:2 * EMB])
    sq = jnp.sum(x * x, axis=1, keepdims=True)
    out_ref[...] = x * lax.rsqrt(jnp.maximum(sq, 1e-12))


BS = 2048


def _tc_combine(types2d, data, scat, s2cat, wcat):
    return pl.pallas_call(
        _tc_body,
        grid=(B // BS,),
        in_specs=[
            pl.BlockSpec((BS, 1), lambda i: (i, 0)),
            pl.BlockSpec((BS, 128), lambda i: (i, 0)),
            pl.BlockSpec((2 * EMB_U, 4 * DIM_A), lambda i: (0, 0)),
            pl.BlockSpec((1, 4 * DIM_A), lambda i: (0, 0)),
            pl.BlockSpec((EMB_U, 2 * EMB), lambda i: (0, 0)),
        ],
        out_specs=pl.BlockSpec((BS, EMB), lambda i: (i, 0)),
        out_shape=jax.ShapeDtypeStruct((B, EMB), jnp.float32),
        compiler_params=pltpu.CompilerParams(
            dimension_semantics=("parallel",)),
    )(types2d, data, scat, s2cat, wcat)


def kernel(node_embeddings, node_type_embeddings, trans_weights,
           trans_weights_s1, trans_weights_s2, train_inputs, train_labels,
           train_types, node_neigh):
    del train_labels
    ntt_u = node_type_embeddings.transpose(1, 2, 0).reshape(T * EMB_U,
                                                            NUM_NODES)
    ne_u = jnp.transpose(node_embeddings)
    tab = _tp_tables(ntt_u, ne_u).reshape(NUM_NODES * 8, EMB_U)
    nn_sc = (node_neigh.transpose(2, 0, 1)
             .reshape(NEIGH, NT, 128, T)
             .transpose(0, 1, 3, 2))
    (data,) = _build_sc_gather()(tab, train_inputs, nn_sc)
    types2d = train_types.reshape(B, 1)
    z = jnp.zeros((EMB_U, DIM_A), jnp.float32)
    scat = jnp.concatenate(
        [jnp.concatenate([trans_weights_s1[0], z, trans_weights_s1[1], z],
                         axis=1),
         jnp.concatenate([z, trans_weights_s1[0], z, trans_weights_s1[1]],
                         axis=1)], axis=0)
    s2cat = jnp.concatenate(
        [trans_weights_s2[0, :, 0], trans_weights_s2[0, :, 0],
         trans_weights_s2[1, :, 0], trans_weights_s2[1, :, 0]]
    ).reshape(1, 4 * DIM_A)
    wcat = jnp.concatenate([trans_weights[0], trans_weights[1]],
                           axis=1)
    return _tc_combine(types2d, data, scat, s2cat, wcat)


# speedup vs baseline: 1.5048x; 1.0347x over previous
import functools

import jax
import jax.numpy as jnp
from jax import lax
from jax.experimental import pallas as pl
from jax.experimental.pallas import tpu as pltpu
from jax.experimental.pallas import tpu_sc as plsc

NUM_NODES = 100000
EMB = 64
EMB_U = 16
T = 2
DIM_A = 20
NEIGH = 10
B = 16384

NW = 32
CH = 64
NCH = B // CH
CPW = NCH // NW
ROWS = CH * T * NEIGH
NT = B // 128


def _sc_body(tab_hbm, ti_hbm, nn_hbm, out,
             idx_v, rows_v, comb_v, tidx_v, nerows_v,
             sem_g, sem_ne):
    c = lax.axis_index("c")
    s = lax.axis_index("s")
    wid = s * 2 + c
    g_base = wid * CPW

    gather_descs = [None, None]
    ne_descs = [None, None]

    def stage(g, buf):
        t = g // 2
        h = g % 2
        pltpu.sync_copy(nn_hbm.at[:, t, :, pl.ds(h * CH, CH)], idx_v[buf])

        def _fix(j, carry):
            for k in range(NEIGH):
                for i in range(T):
                    sl = (k, i, pl.ds(j * 16, 16))
                    idx_v[buf][sl] = idx_v[buf][sl] * 8 + (4 + i)
            return carry

        lax.fori_loop(0, CH // 16, _fix, 0)

        gather_descs[buf] = [
            pltpu.async_copy(
                tab_hbm.at[idx_v[buf].at[k, i]],
                rows_v[buf].at[pl.ds((k * T + i) * CH, CH)], sem_g[buf])
            for k in range(NEIGH) for i in range(T)
        ]
        pltpu.sync_copy(ti_hbm.at[pl.ds(g * CH, CH)],
                        tidx_v[buf].at[pl.ds(0, CH)])

        def _fix_ti(j, carry):
            lo = (pl.ds(j * 16, 16),)
            base = tidx_v[buf][lo] * 8
            for q in range(3, 0, -1):
                tidx_v[buf][(pl.ds(q * CH + j * 16, 16),)] = base + q
            tidx_v[buf][lo] = base
            return carry

        lax.fori_loop(0, CH // 16, _fix_ti, 0)
        ne_descs[buf] = [
            pltpu.async_copy(
                tab_hbm.at[tidx_v[buf].at[pl.ds(h2 * CH, CH)]],
                nerows_v[buf].at[pl.ds(h2 * CH, CH)], sem_ne[buf])
            for h2 in range(4)
        ]

    def consume(g, buf):
        for d in gather_descs[buf]:
            d.wait()
        for d in ne_descs[buf]:
            d.wait()

        def _accum(e, carry):
            r0 = rows_v[buf][e, :]
            r1 = rows_v[buf][CH + e, :]
            for k in range(1, NEIGH):
                r0 = r0 + rows_v[buf][(k * T) * CH + e, :]
                r1 = r1 + rows_v[buf][(k * T + 1) * CH + e, :]
            for q in range(4):
                comb_v[e, pl.ds(q * 16, 16)] = nerows_v[buf][q * CH + e, :]
            comb_v[e, pl.ds(64, 16)] = r0 * (1.0 / NEIGH)
            comb_v[e, pl.ds(80, 16)] = r1 * (1.0 / NEIGH)
            return carry

        lax.fori_loop(0, CH, _accum, 0)
        pltpu.sync_copy(comb_v, out.at[pl.ds(g * CH, CH)])

    stage(g_base, 0)
    for q in range(CPW):
        if q + 1 < CPW:
            stage(g_base + q + 1, (q + 1) % 2)
        consume(g_base + q, q % 2)


@functools.cache
def _build_sc_gather():
    return pl.kernel(
        _sc_body,
        out_type=[
            jax.ShapeDtypeStruct((B, 128), jnp.float32),
        ],
        mesh=plsc.VectorSubcoreMesh(core_axis_name="c", subcore_axis_name="s"),
        compiler_params=pltpu.CompilerParams(use_tc_tiling_on_sc=False),
        scratch_types=[
            [pltpu.VMEM((NEIGH, T, CH), jnp.int32) for _ in range(2)],
            [pltpu.VMEM((ROWS, EMB_U), jnp.float32) for _ in range(2)],
            pltpu.VMEM((CH, 128), jnp.float32),
            [pltpu.VMEM((4 * CH,), jnp.int32) for _ in range(2)],
            [pltpu.VMEM((4 * CH, EMB_U), jnp.float32) for _ in range(2)],
            [pltpu.SemaphoreType.DMA for _ in range(2)],
            [pltpu.SemaphoreType.DMA for _ in range(2)],
        ],
    )


VB = 8192
NVB = -(-NUM_NODES // VB)


def _tp_body(xu_ref, xe_ref, o_ref):
    o_ref[...] = jnp.concatenate(
        [xe_ref[...].T, xu_ref[...].T,
         jnp.zeros((VB, 32), jnp.float32)], axis=1)


def _tp_tables(ntt_u, ne_u):
    return pl.pallas_call(
        _tp_body,
        grid=(NVB,),
        in_specs=[
            pl.BlockSpec((T * EMB_U, VB), lambda j: (0, j)),
            pl.BlockSpec((EMB, VB), lambda j: (0, j)),
        ],
        out_specs=pl.BlockSpec((VB, 128), lambda j: (j, 0)),
        out_shape=jax.ShapeDtypeStruct((NUM_NODES, 128), jnp.float32),
        compiler_params=pltpu.CompilerParams(
            dimension_semantics=("parallel",)),
    )(ntt_u, ne_u)


def _tc_body(types_ref, data_ref, scat_ref, s2cat_ref, wcat_ref, out_ref):
    data = data_ref[...]
    ne = data[:, 0:EMB]
    nte0 = data[:, 64:64 + EMB_U]
    nte1 = data[:, 80:80 + EMB_U]
    lhs = data[:, 64:64 + 2 * EMB_U]
    is0 = types_ref[...] == 0
    z = jnp.dot(lhs, scat_ref[...], preferred_element_type=jnp.float32)
    zsel = jnp.where(is0, z[:, 0:2 * DIM_A], z[:, 2 * DIM_A:4 * DIM_A])
    s2sel = jnp.where(is0, s2cat_ref[:, 0:2 * DIM_A],
                      s2cat_ref[:, 2 * DIM_A:4 * DIM_A])
    hs = jnp.tanh(zsel) * s2sel
    l0 = jnp.sum(hs[:, 0:DIM_A], axis=1, keepdims=True)
    l1 = jnp.sum(hs[:, DIM_A:2 * DIM_A], axis=1, keepdims=True)
    a0 = 0.5 * (jnp.tanh(0.5 * (l0 - l1)) + 1.0)
    comb = nte1 + a0 * (nte0 - nte1)
    d = jnp.dot(comb, wcat_ref[...], preferred_element_type=jnp.float32)
    x = ne + jnp.where(is0, d[:, 0:EMB], d[:, EMB:2 * EMB])
    sq = jnp.sum(x * x, axis=1, keepdims=True)
    out_ref[...] = x * lax.rsqrt(jnp.maximum(sq, 1e-12))


BS = 2048


def _tc_combine(types2d, data, scat, s2cat, wcat):
    return pl.pallas_call(
        _tc_body,
        grid=(B // BS,),
        in_specs=[
            pl.BlockSpec((BS, 1), lambda i: (i, 0)),
            pl.BlockSpec((BS, 128), lambda i: (i, 0)),
            pl.BlockSpec((2 * EMB_U, 4 * DIM_A), lambda i: (0, 0)),
            pl.BlockSpec((1, 4 * DIM_A), lambda i: (0, 0)),
            pl.BlockSpec((EMB_U, 2 * EMB), lambda i: (0, 0)),
        ],
        out_specs=pl.BlockSpec((BS, EMB), lambda i: (i, 0)),
        out_shape=jax.ShapeDtypeStruct((B, EMB), jnp.float32),
        compiler_params=pltpu.CompilerParams(
            dimension_semantics=("parallel",)),
    )(types2d, data, scat, s2cat, wcat)


def kernel(node_embeddings, node_type_embeddings, trans_weights,
           trans_weights_s1, trans_weights_s2, train_inputs, train_labels,
           train_types, node_neigh):
    del train_labels
    ntt_u = node_type_embeddings.transpose(1, 2, 0).reshape(T * EMB_U,
                                                            NUM_NODES)
    ne_u = jnp.transpose(node_embeddings)
    tab = _tp_tables(ntt_u, ne_u).reshape(NUM_NODES * 8, EMB_U)
    nn_sc = (node_neigh.transpose(2, 0, 1)
             .reshape(NEIGH, NT, 128, T)
             .transpose(0, 1, 3, 2))
    (data,) = _build_sc_gather()(tab, train_inputs, nn_sc)
    types2d = train_types.reshape(B, 1)
    z = jnp.zeros((EMB_U, DIM_A), jnp.float32)
    scat = jnp.concatenate(
        [jnp.concatenate([trans_weights_s1[0], z, trans_weights_s1[1], z],
                         axis=1),
         jnp.concatenate([z, trans_weights_s1[0], z, trans_weights_s1[1]],
                         axis=1)], axis=0)
    s2cat = jnp.concatenate(
        [trans_weights_s2[0, :, 0], trans_weights_s2[0, :, 0],
         trans_weights_s2[1, :, 0], trans_weights_s2[1, :, 0]]
    ).reshape(1, 4 * DIM_A)
    wcat = jnp.concatenate([trans_weights[0], trans_weights[1]],
                           axis=1)
    return _tc_combine(types2d, data, scat, s2cat, wcat)
